# Initial kernel scaffold; baseline (speedup 1.0000x reference)
import jax
import jax.numpy as jnp
from jax.experimental import pallas as pl


def kernel(x, edge_index):
    raise NotImplementedError("write your pallas kernel here")


# speedup vs baseline: 3.9411x; 3.9411x over previous
import functools

import jax
import jax.numpy as jnp
from jax import lax
from jax.experimental import pallas as pl
from jax.experimental.pallas import tpu as pltpu
from jax.experimental.pallas import tpu_sc as plsc

NODES = 10000
EDGES = 320000
FEAT = 128

NC, NS = 2, 16
NW = NC * NS
CH = 128
EPW = EDGES // NW
CPW = -(-EPW // CH)
CPW = CPW + (CPW % 2)
PAD = CPW * CH - EPW
RPT = 632
NODES_PAD = NS * RPT
DEPTH = 2
SEG = 2
CPS = CPW // SEG


def _sc_body(x_hbm, src_hbm, tgt_hbm, zero_hbm, out0, out1,
             acc, sidx, tidx, r0, r1, s0, s1):
    c = lax.axis_index("c")
    s = lax.axis_index("s")
    wid = s * NC + c

    row0 = s * RPT
    pltpu.sync_copy(zero_hbm.at[pl.ds(row0, RPT)], acc.at[pl.ds(row0, RPT)])

    plsc.subcore_barrier()

    cbase = wid * CPW
    rows = (r0, r1)
    sems = (s0, s1)

    for seg in range(SEG):
        pltpu.sync_copy(src_hbm.at[pl.ds(cbase + seg * CPS, CPS)], sidx)
        pltpu.sync_copy(tgt_hbm.at[pl.ds(cbase + seg * CPS, CPS)], tidx)

        def body(j, carry):
            k = j * DEPTH
            dmas = []
            for b in range(DEPTH):
                dmas.append(
                    pltpu.async_copy(x_hbm.at[sidx.at[k + b]], rows[b],
                                     sems[b]))
            for b in range(DEPTH):
                dmas[b].wait()
                pltpu.sync_copy(rows[b], acc.at[tidx.at[k + b]], add=True)
            return carry

        lax.fori_loop(0, CPS // DEPTH, body, 0)
    plsc.subcore_barrier()

    @pl.when(c == 0)
    def _():
        pltpu.sync_copy(acc.at[pl.ds(row0, RPT)], out0.at[pl.ds(row0, RPT)])

    @pl.when(c == 1)
    def _():
        pltpu.sync_copy(acc.at[pl.ds(row0, RPT)], out1.at[pl.ds(row0, RPT)])


_sc_scatter = functools.partial(
    pl.kernel,
    mesh=plsc.VectorSubcoreMesh(core_axis_name="c", subcore_axis_name="s"),
    out_type=(
        jax.ShapeDtypeStruct((NODES_PAD, FEAT), jnp.float32),
        jax.ShapeDtypeStruct((NODES_PAD, FEAT), jnp.float32),
    ),
    scratch_types=[
        pltpu.VMEM_SHARED((NODES_PAD, FEAT), jnp.float32),
        pltpu.VMEM((CPS, CH), jnp.int32),
        pltpu.VMEM((CPS, CH), jnp.int32),
        pltpu.VMEM((CH, FEAT), jnp.float32),
        pltpu.VMEM((CH, FEAT), jnp.float32),
        pltpu.SemaphoreType.DMA,
        pltpu.SemaphoreType.DMA,
    ],
)(_sc_body)


def _add_body(a_ref, b_ref, o_ref):
    o_ref[...] = a_ref[...] + b_ref[...]


_tc_add = pl.pallas_call(
    _add_body,
    grid=(NS,),
    in_specs=[
        pl.BlockSpec((RPT, FEAT), lambda i: (i, 0)),
        pl.BlockSpec((RPT, FEAT), lambda i: (i, 0)),
    ],
    out_specs=pl.BlockSpec((RPT, FEAT), lambda i: (i, 0)),
    out_shape=jax.ShapeDtypeStruct((NODES_PAD, FEAT), jnp.float32),
)


def kernel(x, edge_index):
    src = edge_index[0].reshape(NW, EPW)
    tgt = edge_index[1].reshape(NW, EPW)
    src = jnp.pad(src, ((0, 0), (0, PAD))).reshape(NW * CPW, CH)
    tpad = NODES + jnp.arange(PAD, dtype=jnp.int32) % 16
    tgt = jnp.concatenate(
        [tgt, jnp.broadcast_to(tpad, (NW, PAD))], axis=1
    ).reshape(NW * CPW, CH)
    zero = jnp.zeros((NODES_PAD, FEAT), jnp.float32)
    p0, p1 = _sc_scatter(x, src, tgt, zero)
    return _tc_add(p0, p1)[:NODES]


# speedup vs baseline: 3.9955x; 1.0138x over previous
import functools

import jax
import jax.numpy as jnp
from jax import lax
from jax.experimental import pallas as pl
from jax.experimental.pallas import tpu as pltpu
from jax.experimental.pallas import tpu_sc as plsc

NODES = 10000
EDGES = 320000
FEAT = 128

NC, NS = 2, 16
NW = NC * NS
CH = 128
EPW = EDGES // NW
CPW = -(-EPW // CH)
CPW = CPW + (CPW % 2)
PAD = CPW * CH - EPW
RPT = 632
NODES_PAD = NS * RPT
DEPTH = 2
SEG = 2
CPS = CPW // SEG


def _sc_body(x_hbm, src_hbm, tgt_hbm, zero_hbm, out0, out1,
             acc, sidx, tidx, r0, r1, g0, g1, t0, t1):
    gsems = (g0, g1)
    ssems = (t0, t1)
    c = lax.axis_index("c")
    s = lax.axis_index("s")
    wid = s * NC + c

    row0 = s * RPT
    pltpu.sync_copy(zero_hbm.at[pl.ds(row0, RPT)], acc.at[pl.ds(row0, RPT)])

    plsc.subcore_barrier()

    cbase = wid * CPW
    rows = (r0, r1)

    for seg in range(SEG):
        pltpu.sync_copy(src_hbm.at[pl.ds(cbase + seg * CPS, CPS)], sidx)
        pltpu.sync_copy(tgt_hbm.at[pl.ds(cbase + seg * CPS, CPS)], tidx)

        def body(j, carry):
            k = j * DEPTH
            for b in range(DEPTH):
                @pl.when(j > 0)
                def _(b=b, k=k):
                    pltpu.make_async_copy(
                        rows[b], acc.at[tidx.at[k + b - DEPTH]],
                        ssems[b]).wait()
                pltpu.async_copy(x_hbm.at[sidx.at[k + b]], rows[b], gsems[b])
            for b in range(DEPTH):
                pltpu.make_async_copy(x_hbm.at[sidx.at[k + b]], rows[b],
                                      gsems[b]).wait()
                pltpu.async_copy(rows[b], acc.at[tidx.at[k + b]], ssems[b],
                                 add=True)
            return carry

        lax.fori_loop(0, CPS // DEPTH, body, 0)
        for b in range(DEPTH):
            pltpu.make_async_copy(rows[b], acc.at[tidx.at[CPS - DEPTH + b]],
                                  ssems[b]).wait()
    plsc.subcore_barrier()

    @pl.when(c == 0)
    def _():
        pltpu.sync_copy(acc.at[pl.ds(row0, RPT)], out0.at[pl.ds(row0, RPT)])

    @pl.when(c == 1)
    def _():
        pltpu.sync_copy(acc.at[pl.ds(row0, RPT)], out1.at[pl.ds(row0, RPT)])


_sc_scatter = functools.partial(
    pl.kernel,
    mesh=plsc.VectorSubcoreMesh(core_axis_name="c", subcore_axis_name="s"),
    out_type=(
        jax.ShapeDtypeStruct((NODES_PAD, FEAT), jnp.float32),
        jax.ShapeDtypeStruct((NODES_PAD, FEAT), jnp.float32),
    ),
    scratch_types=[
        pltpu.VMEM_SHARED((NODES_PAD, FEAT), jnp.float32),
        pltpu.VMEM((CPS, CH), jnp.int32),
        pltpu.VMEM((CPS, CH), jnp.int32),
        pltpu.VMEM((CH, FEAT), jnp.float32),
        pltpu.VMEM((CH, FEAT), jnp.float32),
        pltpu.SemaphoreType.DMA,
        pltpu.SemaphoreType.DMA,
        pltpu.SemaphoreType.DMA,
        pltpu.SemaphoreType.DMA,
    ],
)(_sc_body)


def _add_body(a_ref, b_ref, o_ref):
    o_ref[...] = a_ref[...] + b_ref[...]


_tc_add = pl.pallas_call(
    _add_body,
    grid=(NS,),
    in_specs=[
        pl.BlockSpec((RPT, FEAT), lambda i: (i, 0)),
        pl.BlockSpec((RPT, FEAT), lambda i: (i, 0)),
    ],
    out_specs=pl.BlockSpec((RPT, FEAT), lambda i: (i, 0)),
    out_shape=jax.ShapeDtypeStruct((NODES_PAD, FEAT), jnp.float32),
)


def kernel(x, edge_index):
    src = edge_index[0].reshape(NW, EPW)
    tgt = edge_index[1].reshape(NW, EPW)
    src = jnp.pad(src, ((0, 0), (0, PAD))).reshape(NW * CPW, CH)
    tpad = NODES + jnp.arange(PAD, dtype=jnp.int32) % 16
    tgt = jnp.concatenate(
        [tgt, jnp.broadcast_to(tpad, (NW, PAD))], axis=1
    ).reshape(NW * CPW, CH)
    zero = jnp.zeros((NODES_PAD, FEAT), jnp.float32)
    p0, p1 = _sc_scatter(x, src, tgt, zero)
    return _tc_add(p0, p1)[:NODES]


# speedup vs baseline: 4.1791x; 1.0459x over previous
import functools

import jax
import jax.numpy as jnp
from jax import lax
from jax.experimental import pallas as pl
from jax.experimental.pallas import tpu as pltpu
from jax.experimental.pallas import tpu_sc as plsc

NODES = 10000
EDGES = 320000
FEAT = 128

NC, NS = 2, 16
NW = NC * NS
CH = 64
EPW = EDGES // NW
DEPTH = 4
CPW = -(-EPW // CH)
CPW = -(-CPW // DEPTH) * DEPTH
PAD = CPW * CH - EPW
RPT = 632
NODES_PAD = NS * RPT
SEG = 4
CPS = CPW // SEG


def _sc_body(x_hbm, src_hbm, tgt_hbm, zero_hbm, out0, out1,
             acc, sidx, tidx, *bufs):
    rows = bufs[:DEPTH]
    gsems = bufs[DEPTH:2 * DEPTH]
    ssems = bufs[2 * DEPTH:3 * DEPTH]
    c = lax.axis_index("c")
    s = lax.axis_index("s")
    wid = s * NC + c

    row0 = s * RPT
    pltpu.sync_copy(zero_hbm.at[pl.ds(row0, RPT)], acc.at[pl.ds(row0, RPT)])

    plsc.subcore_barrier()

    cbase = wid * CPW

    for seg in range(SEG):
        pltpu.sync_copy(src_hbm.at[pl.ds(cbase + seg * CPS, CPS)], sidx)
        pltpu.sync_copy(tgt_hbm.at[pl.ds(cbase + seg * CPS, CPS)], tidx)

        def body(j, carry):
            k = j * DEPTH
            for b in range(DEPTH):
                @pl.when(j > 0)
                def _(b=b, k=k):
                    pltpu.make_async_copy(
                        rows[b], acc.at[tidx.at[k + b - DEPTH]],
                        ssems[b]).wait()
                pltpu.async_copy(x_hbm.at[sidx.at[k + b]], rows[b], gsems[b])
            for b in range(DEPTH):
                pltpu.make_async_copy(x_hbm.at[sidx.at[k + b]], rows[b],
                                      gsems[b]).wait()
                pltpu.async_copy(rows[b], acc.at[tidx.at[k + b]], ssems[b],
                                 add=True)
            return carry

        lax.fori_loop(0, CPS // DEPTH, body, 0)
        for b in range(DEPTH):
            pltpu.make_async_copy(
                rows[b], acc.at[tidx.at[CPS - DEPTH + b]], ssems[b]).wait()
    plsc.subcore_barrier()

    @pl.when(c == 0)
    def _():
        pltpu.sync_copy(acc.at[pl.ds(row0, RPT)], out0.at[pl.ds(row0, RPT)])

    @pl.when(c == 1)
    def _():
        pltpu.sync_copy(acc.at[pl.ds(row0, RPT)], out1.at[pl.ds(row0, RPT)])


_sc_scatter = functools.partial(
    pl.kernel,
    mesh=plsc.VectorSubcoreMesh(core_axis_name="c", subcore_axis_name="s"),
    out_type=(
        jax.ShapeDtypeStruct((NODES_PAD, FEAT), jnp.float32),
        jax.ShapeDtypeStruct((NODES_PAD, FEAT), jnp.float32),
    ),
    scratch_types=[
        pltpu.VMEM_SHARED((NODES_PAD, FEAT), jnp.float32),
        pltpu.VMEM((CPS, CH), jnp.int32),
        pltpu.VMEM((CPS, CH), jnp.int32),
    ] + [pltpu.VMEM((CH, FEAT), jnp.float32) for _ in range(DEPTH)]
      + [pltpu.SemaphoreType.DMA for _ in range(2 * DEPTH)],
)(_sc_body)


def _add_body(a_ref, b_ref, o_ref):
    o_ref[...] = a_ref[...] + b_ref[...]


_tc_add = pl.pallas_call(
    _add_body,
    grid=(NS,),
    in_specs=[
        pl.BlockSpec((RPT, FEAT), lambda i: (i, 0)),
        pl.BlockSpec((RPT, FEAT), lambda i: (i, 0)),
    ],
    out_specs=pl.BlockSpec((RPT, FEAT), lambda i: (i, 0)),
    out_shape=jax.ShapeDtypeStruct((NODES_PAD, FEAT), jnp.float32),
)


def kernel(x, edge_index):
    src = edge_index[0].reshape(NW, EPW)
    tgt = edge_index[1].reshape(NW, EPW)
    src = jnp.pad(src, ((0, 0), (0, PAD))).reshape(NW * CPW, CH)
    tpad = NODES + jnp.arange(PAD, dtype=jnp.int32) % 16
    tgt = jnp.concatenate(
        [tgt, jnp.broadcast_to(tpad, (NW, PAD))], axis=1
    ).reshape(NW * CPW, CH)
    zero = jnp.zeros((NODES_PAD, FEAT), jnp.float32)
    p0, p1 = _sc_scatter(x, src, tgt, zero)
    return _tc_add(p0, p1)[:NODES]
